# Initial kernel scaffold; baseline (speedup 1.0000x reference)
import jax
import jax.numpy as jnp
from jax.experimental import pallas as pl


def kernel(objects_list, context, W0, b0, W1, b1, Wkv, bkv, Wq, bq):
    raise NotImplementedError("write your pallas kernel here")


# speedup vs baseline: 7.5365x; 7.5365x over previous
import math

import jax
import jax.numpy as jnp
from jax.experimental import pallas as pl


def _fused_body(x_ref, ctx_ref, W0_ref, b0_ref, W1_ref, b1_ref,
                Wkv_ref, bkv_ref, Wq_ref, bq_ref,
                emb_ref, w_ref):
    H = W1_ref.shape[1]
    x = x_ref[0]
    h = jnp.maximum(jnp.dot(x, W0_ref[...], preferred_element_type=jnp.float32)
                    + b0_ref[...], 0.0)
    h = jnp.maximum(jnp.dot(h, W1_ref[...], preferred_element_type=jnp.float32)
                    + b1_ref[...], 0.0)
    kv = jnp.dot(h, Wkv_ref[...], preferred_element_type=jnp.float32) + bkv_ref[...]
    key = kv[:, :H]
    value = kv[:, H:]
    q = jnp.dot(ctx_ref[0], Wq_ref[...], preferred_element_type=jnp.float32) \
        + bq_ref[...]
    logits = jnp.dot(key, q.T,
                     preferred_element_type=jnp.float32) * (1.0 / math.sqrt(H))
    m = jnp.max(logits)
    ex = jnp.exp(logits - m)
    s = jnp.sum(ex)
    w = ex / s
    emb_ref[0] = jnp.dot(w.T, value, preferred_element_type=jnp.float32)
    w_ref[0] = w.T


def kernel(objects_list, context, W0, b0, W1, b1, Wkv, bkv, Wq, bq):
    B, L, D = objects_list.shape
    D_CTX = context.shape[1]
    H = W1.shape[1]
    ctx3 = context.reshape(B, 1, D_CTX)
    b0r = b0.reshape(1, -1)
    b1r = b1.reshape(1, -1)
    bkvr = bkv.reshape(1, -1)
    bqr = bq.reshape(1, -1)

    full = lambda shape: pl.BlockSpec(shape, lambda b: (0,) * len(shape))
    emb, w = pl.pallas_call(
        _fused_body,
        grid=(B,),
        in_specs=[
            pl.BlockSpec((1, L, D), lambda b: (b, 0, 0)),
            pl.BlockSpec((1, 1, D_CTX), lambda b: (b, 0, 0)),
            full(W0.shape), full(b0r.shape),
            full(W1.shape), full(b1r.shape),
            full(Wkv.shape), full(bkvr.shape),
            full(Wq.shape), full(bqr.shape),
        ],
        out_specs=[
            pl.BlockSpec((1, 1, H), lambda b: (b, 0, 0)),
            pl.BlockSpec((1, 1, L), lambda b: (b, 0, 0)),
        ],
        out_shape=[
            jax.ShapeDtypeStruct((B, 1, H), jnp.float32),
            jax.ShapeDtypeStruct((B, 1, L), jnp.float32),
        ],
    )(objects_list, ctx3, W0, b0r, W1, b1r, Wkv, bkvr, Wq, bqr)
    return emb.reshape(B, H), w.reshape(B, L)
